# Initial kernel scaffold; baseline (speedup 1.0000x reference)
import jax
import jax.numpy as jnp
from jax.experimental import pallas as pl


def kernel(states, worker_ids, worker_cov):
    raise NotImplementedError("write your pallas kernel here")


# speedup vs baseline: 1.9350x; 1.9350x over previous
import functools

import jax
import jax.numpy as jnp
from jax import lax
from jax.experimental import pallas as pl
from jax.experimental.pallas import tpu as pltpu
from jax.experimental.pallas import tpu_sc as plsc

NC = 2
NS = 16
NW = NC * NS
LANES = 16
CHUNK = 128

BATCH = 16384
ACTION_DIM = 64
PAD_ROWS = 128
ROWS_PER_TILE = PAD_ROWS // NS
B_PER_W = BATCH // NW
N_CHUNKS = B_PER_W // CHUNK


@functools.partial(
    pl.kernel,
    out_type=jax.ShapeDtypeStruct((BATCH, ACTION_DIM), jnp.float32),
    mesh=plsc.VectorSubcoreMesh(core_axis_name="c", subcore_axis_name="s"),
    compiler_params=pltpu.CompilerParams(use_tc_tiling_on_sc=False),
    scratch_types=[
        pltpu.VMEM((N_CHUNKS, CHUNK), jnp.int32),
        pltpu.VMEM((ROWS_PER_TILE, ACTION_DIM), jnp.float32),
        pltpu.VMEM((B_PER_W, ACTION_DIM), jnp.float32),
        pltpu.VMEM_SHARED((PAD_ROWS, ACTION_DIM), jnp.float32),
        pltpu.SemaphoreType.DMA,
    ],
)
def _gather_exp(table_hbm, idx_hbm, out_hbm, idx_v, tbl_v, rows_v, tbl_sh, sem):
    sid = lax.axis_index("s")
    wid = sid * NC + lax.axis_index("c")
    base = wid * B_PER_W

    pltpu.sync_copy(table_hbm.at[pl.ds(sid * ROWS_PER_TILE, ROWS_PER_TILE)], tbl_v)
    for r in range(ROWS_PER_TILE):
        for j in range(ACTION_DIM // LANES):
            v = tbl_v[r, pl.ds(j * LANES, LANES)]
            tbl_v[r, pl.ds(j * LANES, LANES)] = jnp.exp(v) + 1e-8
    pltpu.sync_copy(tbl_v, tbl_sh.at[pl.ds(sid * ROWS_PER_TILE, ROWS_PER_TILE)])
    pltpu.sync_copy(idx_hbm.at[wid], idx_v)
    plsc.subcore_barrier()

    gathers = [
        pltpu.async_copy(
            tbl_sh.at[idx_v.at[j]],
            rows_v.at[pl.ds(j * CHUNK, CHUNK)],
            sem,
        )
        for j in range(N_CHUNKS)
    ]
    for g in gathers:
        g.wait()
    pltpu.sync_copy(rows_v, out_hbm.at[pl.ds(base, B_PER_W)])


def kernel(states, worker_ids, worker_cov):
    del states
    table = jnp.pad(worker_cov.T, ((0, PAD_ROWS - worker_cov.shape[1]), (0, 0)))
    idx = worker_ids.astype(jnp.int32).reshape(NW, N_CHUNKS, CHUNK)
    out_cov = _gather_exp(table, idx)
    mu = jnp.zeros((BATCH, ACTION_DIM), dtype=jnp.float32)
    return (out_cov, mu)
